# Initial kernel scaffold; baseline (speedup 1.0000x reference)
import jax
import jax.numpy as jnp
from jax.experimental import pallas as pl


def kernel(x, edge_index, conv1_W1, conv1_b1, conv1_W2, conv1_b2, conv2_W1, conv2_b1, conv2_W2, conv2_b2, conv3_W1, conv3_b1, conv3_W2, conv3_b2, lin1_W, lin1_b, lin2_W, lin2_b):
    raise NotImplementedError("write your pallas kernel here")


# speedup vs baseline: 2.8736x; 2.8736x over previous
import functools

import jax
import jax.numpy as jnp
from jax import lax
from jax.experimental import pallas as pl
from jax.experimental.pallas import tpu as pltpu
from jax.experimental.pallas import tpu_sc as plsc

N = 10000
D = 256
HALF = 128
NSUB = 16
CHUNK = 128
ACC_ROWS = 10240
ZERO_ROWS = ACC_ROWS // NSUB


def _sc_scatter_add(x_flat, src2, dst_p, zeros_blk, e_pad):
    chunks_per_sub = e_pad // (NSUB * CHUNK)
    span = e_pad // NSUB

    mesh = plsc.VectorSubcoreMesh(core_axis_name="c", subcore_axis_name="s")

    @functools.partial(
        pl.kernel,
        out_type=jax.ShapeDtypeStruct((2 * ACC_ROWS, HALF), jnp.float32),
        mesh=mesh,
        scratch_types=[
            pltpu.VMEM_SHARED((ACC_ROWS, HALF), jnp.float32),
            pltpu.VMEM((CHUNK,), jnp.int32),
            pltpu.VMEM((CHUNK,), jnp.int32),
            pltpu.VMEM((CHUNK, HALF), jnp.float32),
            pltpu.SemaphoreType.DMA,
        ],
    )
    def sc_kernel(x_hbm, src_hbm, dst_hbm, zero_hbm, out_hbm,
                  acc, sidx, didx, rows, sem):
        c = lax.axis_index("c")
        s = lax.axis_index("s")
        pltpu.sync_copy(zero_hbm, acc.at[pl.ds(s * ZERO_ROWS, ZERO_ROWS)])
        plsc.subcore_barrier()

        sbase = c * e_pad + s * span
        dbase = s * span

        @pl.loop(0, chunks_per_sub)
        def _(i):
            off = i * CHUNK
            pltpu.sync_copy(src_hbm.at[pl.ds(sbase + off, CHUNK)], sidx)
            pltpu.sync_copy(dst_hbm.at[pl.ds(dbase + off, CHUNK)], didx)
            pltpu.async_copy(x_hbm.at[sidx], rows, sem).wait()
            pltpu.sync_copy(rows, acc.at[didx], add=True)

        plsc.subcore_barrier()
        out0 = c * ACC_ROWS + s * ZERO_ROWS
        pltpu.sync_copy(acc.at[pl.ds(s * ZERO_ROWS, ZERO_ROWS)],
                        out_hbm.at[pl.ds(out0, ZERO_ROWS)])

    return sc_kernel(x_flat, src2, dst_p, zeros_blk)


def _tc_layer_body(h, a, W1, b1, W2, b2, o, pool):
    u = jnp.concatenate([h[0] + a[0], h[1] + a[1]], axis=1)
    t = jnp.dot(u, W1[...], preferred_element_type=jnp.float32,
                precision=lax.Precision.HIGHEST) + b1[...]
    t = jnp.maximum(t, 0.0)
    v = jnp.dot(t, W2[...], preferred_element_type=jnp.float32,
                precision=lax.Precision.HIGHEST) + b2[...]
    v = jnp.maximum(v, 0.0)
    o[0, :, :] = v[:, :HALF]
    o[1, :, :] = v[:, HALF:]

    @pl.when(pl.program_id(0) == 0)
    def _():
        pool[...] = jnp.zeros_like(pool)

    pool[...] += jnp.sum(v, axis=0, keepdims=True)


def _tc_layer(h_s, agg_s, W1, b1, W2, b2):
    R = 1000
    out = pl.pallas_call(
        _tc_layer_body,
        grid=(N // R,),
        in_specs=[
            pl.BlockSpec((2, R, HALF), lambda i: (0, i, 0)),
            pl.BlockSpec((2, R, HALF), lambda i: (0, i, 0)),
            pl.BlockSpec((D, D), lambda i: (0, 0)),
            pl.BlockSpec((1, D), lambda i: (0, 0)),
            pl.BlockSpec((D, D), lambda i: (0, 0)),
            pl.BlockSpec((1, D), lambda i: (0, 0)),
        ],
        out_specs=[
            pl.BlockSpec((2, R, HALF), lambda i: (0, i, 0)),
            pl.BlockSpec((1, D), lambda i: (0, 0)),
        ],
        out_shape=[
            jax.ShapeDtypeStruct((2, N, HALF), jnp.float32),
            jax.ShapeDtypeStruct((1, D), jnp.float32),
        ],
    )(h_s, agg_s, W1, b1.reshape(1, D), W2, b2.reshape(1, D))
    return out


def _tc_head_body(p1, p2, p3, w1, bb1, w2, bb2, o):
    p = jnp.concatenate([p1[...], p2[...], p3[...]], axis=1)
    hh = jnp.dot(p, w1[...], preferred_element_type=jnp.float32,
                 precision=lax.Precision.HIGHEST) + bb1[...]
    hh = jnp.maximum(hh, 0.0)
    o[...] = jnp.dot(hh, w2[...], preferred_element_type=jnp.float32,
                     precision=lax.Precision.HIGHEST) + bb2[...]


def _tc_head(p1, p2, p3, lin1_W, lin1_b, lin2_Wp, lin2_bp):
    return pl.pallas_call(
        _tc_head_body,
        out_shape=jax.ShapeDtypeStruct((1, 128), jnp.float32),
    )(p1, p2, p3, lin1_W, lin1_b.reshape(1, -1), lin2_Wp, lin2_bp)


def kernel(x, edge_index, conv1_W1, conv1_b1, conv1_W2, conv1_b2,
           conv2_W1, conv2_b1, conv2_W2, conv2_b2,
           conv3_W1, conv3_b1, conv3_W2, conv3_b2,
           lin1_W, lin1_b, lin2_W, lin2_b):
    src = edge_index[0]
    dst = edge_index[1]
    E = src.shape[0]
    epc = NSUB * CHUNK
    e_pad = ((E + epc - 1) // epc) * epc
    pad = e_pad - E
    src_p = jnp.concatenate([src, jnp.zeros((pad,), jnp.int32)])
    dst_p = jnp.concatenate([dst, jnp.full((pad,), N, jnp.int32)])
    src2 = jnp.concatenate([src_p, src_p + N])
    zeros_blk = jnp.zeros((ZERO_ROWS, HALF), jnp.float32)

    h = jnp.stack([x[:, :HALF], x[:, HALF:]])

    pools = []
    for (W1, b1, W2, b2) in (
        (conv1_W1, conv1_b1, conv1_W2, conv1_b2),
        (conv2_W1, conv2_b1, conv2_W2, conv2_b2),
        (conv3_W1, conv3_b1, conv3_W2, conv3_b2),
    ):
        agg = _sc_scatter_add(h.reshape(2 * N, HALF), src2, dst_p,
                              zeros_blk, e_pad)
        agg = agg.reshape(2, ACC_ROWS, HALF)[:, :N, :]
        h, p = _tc_layer(h, agg, W1, b1, W2, b2)
        pools.append(p)

    C = lin2_W.shape[1]
    lin2_Wp = jnp.zeros((lin2_W.shape[0], 128), jnp.float32).at[:, :C].set(lin2_W)
    lin2_bp = jnp.zeros((1, 128), jnp.float32).at[:, :C].set(lin2_b)
    out = _tc_head(pools[0], pools[1], pools[2], lin1_W, lin1_b,
                   lin2_Wp, lin2_bp)
    return out[:, :C]


# speedup vs baseline: 2.8744x; 1.0003x over previous
import functools

import jax
import jax.numpy as jnp
from jax import lax
from jax.experimental import pallas as pl
from jax.experimental.pallas import tpu as pltpu
from jax.experimental.pallas import tpu_sc as plsc

N = 10000
D = 256
HALF = 128
NSUB = 16
CHUNK = 128
NPHASE = 2
ACC_ROWS = 10240
ZERO_ROWS = ACC_ROWS // NSUB


def _sc_scatter_add(x_flat, src2d, dst2d, zeros_blk, e_pad):
    nchunks = e_pad // (NSUB * CHUNK)
    cpp = nchunks // NPHASE
    npairs = cpp // 2

    mesh = plsc.VectorSubcoreMesh(core_axis_name="c", subcore_axis_name="s")

    @functools.partial(
        pl.kernel,
        out_type=jax.ShapeDtypeStruct((2 * ACC_ROWS, HALF), jnp.float32),
        mesh=mesh,
        scratch_types=[
            pltpu.VMEM_SHARED((ACC_ROWS, HALF), jnp.float32),
            pltpu.VMEM((cpp, CHUNK), jnp.int32),
            pltpu.VMEM((cpp, CHUNK), jnp.int32),
            pltpu.VMEM((CHUNK, HALF), jnp.float32),
            pltpu.VMEM((CHUNK, HALF), jnp.float32),
            pltpu.SemaphoreType.DMA,
            pltpu.SemaphoreType.DMA,
        ],
    )
    def sc_kernel(x_hbm, src_hbm, dst_hbm, zero_hbm, out_hbm,
                  acc, sidx, didx, buf0, buf1, sem0, sem1):
        c = lax.axis_index("c")
        s = lax.axis_index("s")
        pltpu.sync_copy(zero_hbm, acc.at[pl.ds(s * ZERO_ROWS, ZERO_ROWS)])
        plsc.subcore_barrier()

        @pl.loop(0, NPHASE)
        def _(ph):
            srow = (c * NSUB + s) * nchunks + ph * cpp
            drow = s * nchunks + ph * cpp
            pltpu.sync_copy(src_hbm.at[pl.ds(srow, cpp)], sidx)
            pltpu.sync_copy(dst_hbm.at[pl.ds(drow, cpp)], didx)

            pltpu.async_copy(x_hbm.at[sidx.at[0]], buf0, sem0)

            @pl.loop(0, npairs)
            def _(p):
                i0 = 2 * p
                pltpu.async_copy(x_hbm.at[sidx.at[i0 + 1]], buf1, sem1)
                pltpu.make_async_copy(x_hbm.at[sidx.at[i0]], buf0, sem0).wait()
                pltpu.sync_copy(buf0, acc.at[didx.at[i0]], add=True)

                @pl.when(i0 + 2 < cpp)
                def _():
                    pltpu.async_copy(x_hbm.at[sidx.at[i0 + 2]], buf0, sem0)

                pltpu.make_async_copy(x_hbm.at[sidx.at[i0 + 1]], buf1,
                                      sem1).wait()
                pltpu.sync_copy(buf1, acc.at[didx.at[i0 + 1]], add=True)

        plsc.subcore_barrier()
        out0 = c * ACC_ROWS + s * ZERO_ROWS
        pltpu.sync_copy(acc.at[pl.ds(s * ZERO_ROWS, ZERO_ROWS)],
                        out_hbm.at[pl.ds(out0, ZERO_ROWS)])

    return sc_kernel(x_flat, src2d, dst2d, zeros_blk)


def _tc_layer_body(h, a, W1, b1, W2, b2, o, pool):
    u = jnp.concatenate([h[0] + a[0], h[1] + a[1]], axis=1)
    t = jnp.dot(u, W1[...], preferred_element_type=jnp.float32,
                precision=lax.Precision.HIGHEST) + b1[...]
    t = jnp.maximum(t, 0.0)
    v = jnp.dot(t, W2[...], preferred_element_type=jnp.float32,
                precision=lax.Precision.HIGHEST) + b2[...]
    v = jnp.maximum(v, 0.0)
    o[0, :, :] = v[:, :HALF]
    o[1, :, :] = v[:, HALF:]

    @pl.when(pl.program_id(0) == 0)
    def _():
        pool[...] = jnp.zeros_like(pool)

    pool[...] += jnp.sum(v, axis=0, keepdims=True)


def _tc_layer(h_s, agg_s, W1, b1, W2, b2):
    R = 1000
    out = pl.pallas_call(
        _tc_layer_body,
        grid=(N // R,),
        in_specs=[
            pl.BlockSpec((2, R, HALF), lambda i: (0, i, 0)),
            pl.BlockSpec((2, R, HALF), lambda i: (0, i, 0)),
            pl.BlockSpec((D, D), lambda i: (0, 0)),
            pl.BlockSpec((1, D), lambda i: (0, 0)),
            pl.BlockSpec((D, D), lambda i: (0, 0)),
            pl.BlockSpec((1, D), lambda i: (0, 0)),
        ],
        out_specs=[
            pl.BlockSpec((2, R, HALF), lambda i: (0, i, 0)),
            pl.BlockSpec((1, D), lambda i: (0, 0)),
        ],
        out_shape=[
            jax.ShapeDtypeStruct((2, N, HALF), jnp.float32),
            jax.ShapeDtypeStruct((1, D), jnp.float32),
        ],
    )(h_s, agg_s, W1, b1.reshape(1, D), W2, b2.reshape(1, D))
    return out


def _tc_head_body(p1, p2, p3, w1, bb1, w2, bb2, o):
    p = jnp.concatenate([p1[...], p2[...], p3[...]], axis=1)
    hh = jnp.dot(p, w1[...], preferred_element_type=jnp.float32,
                 precision=lax.Precision.HIGHEST) + bb1[...]
    hh = jnp.maximum(hh, 0.0)
    o[...] = jnp.dot(hh, w2[...], preferred_element_type=jnp.float32,
                     precision=lax.Precision.HIGHEST) + bb2[...]


def _tc_head(p1, p2, p3, lin1_W, lin1_b, lin2_Wp, lin2_bp):
    return pl.pallas_call(
        _tc_head_body,
        out_shape=jax.ShapeDtypeStruct((1, 128), jnp.float32),
    )(p1, p2, p3, lin1_W, lin1_b.reshape(1, -1), lin2_Wp, lin2_bp)


def kernel(x, edge_index, conv1_W1, conv1_b1, conv1_W2, conv1_b2,
           conv2_W1, conv2_b1, conv2_W2, conv2_b2,
           conv3_W1, conv3_b1, conv3_W2, conv3_b2,
           lin1_W, lin1_b, lin2_W, lin2_b):
    src = edge_index[0]
    dst = edge_index[1]
    E = src.shape[0]
    epc = 2 * NPHASE * NSUB * CHUNK
    e_pad = ((E + epc - 1) // epc) * epc
    pad = e_pad - E
    src_p = jnp.concatenate([src, jnp.zeros((pad,), jnp.int32)])
    dst_p = jnp.concatenate([dst, jnp.full((pad,), N, jnp.int32)])
    src2d = jnp.concatenate([src_p, src_p + N]).reshape(-1, CHUNK)
    dst2d = dst_p.reshape(-1, CHUNK)
    zeros_blk = jnp.zeros((ZERO_ROWS, HALF), jnp.float32)

    h = jnp.stack([x[:, :HALF], x[:, HALF:]])

    pools = []
    for (W1, b1, W2, b2) in (
        (conv1_W1, conv1_b1, conv1_W2, conv1_b2),
        (conv2_W1, conv2_b1, conv2_W2, conv2_b2),
        (conv3_W1, conv3_b1, conv3_W2, conv3_b2),
    ):
        agg = _sc_scatter_add(h.reshape(2 * N, HALF), src2d, dst2d,
                              zeros_blk, e_pad)
        agg = agg.reshape(2, ACC_ROWS, HALF)[:, :N, :]
        h, p = _tc_layer(h, agg, W1, b1, W2, b2)
        pools.append(p)

    C = lin2_W.shape[1]
    lin2_Wp = jnp.zeros((lin2_W.shape[0], 128), jnp.float32).at[:, :C].set(lin2_W)
    lin2_bp = jnp.zeros((1, 128), jnp.float32).at[:, :C].set(lin2_b)
    out = _tc_head(pools[0], pools[1], pools[2], lin1_W, lin1_b,
                   lin2_Wp, lin2_bp)
    return out[:, :C]


# speedup vs baseline: 2.9258x; 1.0179x over previous
import functools

import jax
import jax.numpy as jnp
from jax import lax
from jax.experimental import pallas as pl
from jax.experimental.pallas import tpu as pltpu
from jax.experimental.pallas import tpu_sc as plsc

N = 10000
D = 256
HALF = 128
NSUB = 16
CHUNK = 128
NPHASE = 2
ACC_ROWS = 10240
ZERO_ROWS = ACC_ROWS // NSUB


def _sc_scatter_add(x_flat, src2d, dst2d, zeros_blk, e_pad):
    nchunks = e_pad // (NSUB * CHUNK)
    cpp = nchunks // NPHASE
    npairs = cpp // 2

    mesh = plsc.VectorSubcoreMesh(core_axis_name="c", subcore_axis_name="s")

    @functools.partial(
        pl.kernel,
        out_type=jax.ShapeDtypeStruct((2 * ACC_ROWS, HALF), jnp.float32),
        mesh=mesh,
        scratch_types=[
            pltpu.VMEM_SHARED((ACC_ROWS, HALF), jnp.float32),
            pltpu.VMEM((cpp, CHUNK), jnp.int32),
            pltpu.VMEM((cpp, CHUNK), jnp.int32),
            pltpu.VMEM((CHUNK, HALF), jnp.float32),
            pltpu.VMEM((CHUNK, HALF), jnp.float32),
            pltpu.SemaphoreType.DMA,
            pltpu.SemaphoreType.DMA,
        ],
    )
    def sc_kernel(x_hbm, src_hbm, dst_hbm, zero_hbm, out_hbm,
                  acc, sidx, didx, buf0, buf1, sem0, sem1):
        c = lax.axis_index("c")
        s = lax.axis_index("s")
        pltpu.sync_copy(zero_hbm, acc.at[pl.ds(s * ZERO_ROWS, ZERO_ROWS)])
        plsc.subcore_barrier()

        @pl.loop(0, NPHASE)
        def _(ph):
            srow = (c * NSUB + s) * nchunks + ph * cpp
            drow = s * nchunks + ph * cpp
            pltpu.sync_copy(src_hbm.at[pl.ds(srow, cpp)], sidx)
            pltpu.sync_copy(dst_hbm.at[pl.ds(drow, cpp)], didx)

            pltpu.async_copy(x_hbm.at[sidx.at[0]], buf0, sem0)

            @pl.loop(0, npairs)
            def _(p):
                i0 = 2 * p
                pltpu.async_copy(x_hbm.at[sidx.at[i0 + 1]], buf1, sem1)
                pltpu.make_async_copy(x_hbm.at[sidx.at[i0]], buf0, sem0).wait()
                pltpu.sync_copy(buf0, acc.at[didx.at[i0]], add=True)

                @pl.when(i0 + 2 < cpp)
                def _():
                    pltpu.async_copy(x_hbm.at[sidx.at[i0 + 2]], buf0, sem0)

                pltpu.make_async_copy(x_hbm.at[sidx.at[i0 + 1]], buf1,
                                      sem1).wait()
                pltpu.sync_copy(buf1, acc.at[didx.at[i0 + 1]], add=True)

        plsc.subcore_barrier()
        out0 = c * ACC_ROWS + s * ZERO_ROWS
        pltpu.sync_copy(acc.at[pl.ds(s * ZERO_ROWS, ZERO_ROWS)],
                        out_hbm.at[pl.ds(out0, ZERO_ROWS)])

    return sc_kernel(x_flat, src2d, dst2d, zeros_blk)


def _mlp_block(h, a, W1, b1, W2, b2):
    u = jnp.concatenate([h[0] + a[0], h[1] + a[1]], axis=1)
    t = jnp.dot(u, W1[...], preferred_element_type=jnp.float32,
                precision=lax.Precision.HIGHEST) + b1[...]
    t = jnp.maximum(t, 0.0)
    v = jnp.dot(t, W2[...], preferred_element_type=jnp.float32,
                precision=lax.Precision.HIGHEST) + b2[...]
    return jnp.maximum(v, 0.0)


def _tc_layer_body(h, a, W1, b1, W2, b2, o, pool):
    v = _mlp_block(h, a, W1, b1, W2, b2)
    o[0, :, :] = v[:, :HALF]
    o[1, :, :] = v[:, HALF:]

    @pl.when(pl.program_id(0) == 0)
    def _():
        pool[...] = jnp.zeros_like(pool)

    pool[...] += jnp.sum(v, axis=0, keepdims=True)


_TC_ROWS = 1000


def _layer_in_specs():
    return [
        pl.BlockSpec((2, _TC_ROWS, HALF), lambda i: (0, i, 0)),
        pl.BlockSpec((2, _TC_ROWS, HALF), lambda i: (0, i, 0)),
        pl.BlockSpec((D, D), lambda i: (0, 0)),
        pl.BlockSpec((1, D), lambda i: (0, 0)),
        pl.BlockSpec((D, D), lambda i: (0, 0)),
        pl.BlockSpec((1, D), lambda i: (0, 0)),
    ]


def _tc_layer(h_s, agg_s, W1, b1, W2, b2):
    return pl.pallas_call(
        _tc_layer_body,
        grid=(N // _TC_ROWS,),
        in_specs=_layer_in_specs(),
        out_specs=[
            pl.BlockSpec((2, _TC_ROWS, HALF), lambda i: (0, i, 0)),
            pl.BlockSpec((1, D), lambda i: (0, 0)),
        ],
        out_shape=[
            jax.ShapeDtypeStruct((2, N, HALF), jnp.float32),
            jax.ShapeDtypeStruct((1, D), jnp.float32),
        ],
    )(h_s, agg_s, W1, b1.reshape(1, D), W2, b2.reshape(1, D))


def _tc_layer3_body(h, a, W1, b1, W2, b2, p1, p2, lw1, lb1, lw2, lb2,
                    pool, o):
    v = _mlp_block(h, a, W1, b1, W2, b2)

    @pl.when(pl.program_id(0) == 0)
    def _():
        pool[...] = jnp.zeros_like(pool)

    pool[...] += jnp.sum(v, axis=0, keepdims=True)

    @pl.when(pl.program_id(0) == N // _TC_ROWS - 1)
    def _():
        p = jnp.concatenate([p1[...], p2[...], pool[...]], axis=1)
        hh = jnp.dot(p, lw1[...], preferred_element_type=jnp.float32,
                     precision=lax.Precision.HIGHEST) + lb1[...]
        hh = jnp.maximum(hh, 0.0)
        o[...] = jnp.dot(hh, lw2[...], preferred_element_type=jnp.float32,
                         precision=lax.Precision.HIGHEST) + lb2[...]


def _tc_layer3(h_s, agg_s, W1, b1, W2, b2, p1, p2,
               lin1_W, lin1_b, lin2_Wp, lin2_bp):
    return pl.pallas_call(
        _tc_layer3_body,
        grid=(N // _TC_ROWS,),
        in_specs=_layer_in_specs() + [
            pl.BlockSpec((1, D), lambda i: (0, 0)),
            pl.BlockSpec((1, D), lambda i: (0, 0)),
            pl.BlockSpec((3 * D, 3 * D), lambda i: (0, 0)),
            pl.BlockSpec((1, 3 * D), lambda i: (0, 0)),
            pl.BlockSpec((3 * D, HALF), lambda i: (0, 0)),
            pl.BlockSpec((1, HALF), lambda i: (0, 0)),
        ],
        out_specs=[
            pl.BlockSpec((1, D), lambda i: (0, 0)),
            pl.BlockSpec((1, HALF), lambda i: (0, 0)),
        ],
        out_shape=[
            jax.ShapeDtypeStruct((1, D), jnp.float32),
            jax.ShapeDtypeStruct((1, HALF), jnp.float32),
        ],
    )(h_s, agg_s, W1, b1.reshape(1, D), W2, b2.reshape(1, D), p1, p2,
      lin1_W, lin1_b.reshape(1, 3 * D), lin2_Wp, lin2_bp)


def kernel(x, edge_index, conv1_W1, conv1_b1, conv1_W2, conv1_b2,
           conv2_W1, conv2_b1, conv2_W2, conv2_b2,
           conv3_W1, conv3_b1, conv3_W2, conv3_b2,
           lin1_W, lin1_b, lin2_W, lin2_b):
    src = edge_index[0]
    dst = edge_index[1]
    E = src.shape[0]
    epc = 2 * NPHASE * NSUB * CHUNK
    e_pad = ((E + epc - 1) // epc) * epc
    pad = e_pad - E
    src_p = jnp.concatenate([src, jnp.zeros((pad,), jnp.int32)])
    dst_p = jnp.concatenate([dst, jnp.full((pad,), N, jnp.int32)])
    src2d = jnp.concatenate([src_p, src_p + N]).reshape(-1, CHUNK)
    dst2d = dst_p.reshape(-1, CHUNK)
    zeros_blk = jnp.zeros((ZERO_ROWS, HALF), jnp.float32)

    h = jnp.stack([x[:, :HALF], x[:, HALF:]])

    pools = []
    for (W1, b1, W2, b2) in (
        (conv1_W1, conv1_b1, conv1_W2, conv1_b2),
        (conv2_W1, conv2_b1, conv2_W2, conv2_b2),
    ):
        agg = _sc_scatter_add(h.reshape(2 * N, HALF), src2d, dst2d,
                              zeros_blk, e_pad)
        h, p = _tc_layer(h, agg.reshape(2, ACC_ROWS, HALF), W1, b1, W2, b2)
        pools.append(p)

    C = lin2_W.shape[1]
    lin2_Wp = jnp.zeros((lin2_W.shape[0], HALF), jnp.float32).at[:, :C].set(lin2_W)
    lin2_bp = jnp.zeros((1, HALF), jnp.float32).at[:, :C].set(lin2_b)
    agg = _sc_scatter_add(h.reshape(2 * N, HALF), src2d, dst2d,
                          zeros_blk, e_pad)
    _, out = _tc_layer3(h, agg.reshape(2, ACC_ROWS, HALF),
                        conv3_W1, conv3_b1, conv3_W2, conv3_b2,
                        pools[0], pools[1], lin1_W, lin1_b, lin2_Wp, lin2_bp)
    return out[:, :C]


# speedup vs baseline: 3.2298x; 1.1039x over previous
import functools

import jax
import jax.numpy as jnp
from jax import lax
from jax.experimental import pallas as pl
from jax.experimental.pallas import tpu as pltpu
from jax.experimental.pallas import tpu_sc as plsc

N = 10000
D = 256
HALF = 128
NSUB = 16
CHUNK = 128
NPHASE = 2
ACC_ROWS = 10240
ZERO_ROWS = ACC_ROWS // NSUB


def _sc_scatter_add(x_flat, src2d, dst2d, zeros_blk, e_pad):
    nchunks = e_pad // (NSUB * CHUNK)
    cpp = nchunks // NPHASE
    npairs = cpp // 2

    mesh = plsc.VectorSubcoreMesh(core_axis_name="c", subcore_axis_name="s")

    @functools.partial(
        pl.kernel,
        out_type=jax.ShapeDtypeStruct((2 * ACC_ROWS, HALF), jnp.float32),
        mesh=mesh,
        scratch_types=[
            pltpu.VMEM_SHARED((ACC_ROWS, HALF), jnp.float32),
            pltpu.VMEM((cpp, CHUNK), jnp.int32),
            pltpu.VMEM((cpp, CHUNK), jnp.int32),
            pltpu.VMEM((CHUNK, HALF), jnp.float32),
            pltpu.VMEM((CHUNK, HALF), jnp.float32),
            pltpu.SemaphoreType.DMA,
            pltpu.SemaphoreType.DMA,
        ],
    )
    def sc_kernel(x_hbm, src_hbm, dst_hbm, zero_hbm, out_hbm,
                  acc, sidx, didx, buf0, buf1, sem0, sem1):
        c = lax.axis_index("c")
        s = lax.axis_index("s")
        pltpu.sync_copy(zero_hbm, acc.at[pl.ds(s * ZERO_ROWS, ZERO_ROWS)])
        plsc.subcore_barrier()

        @pl.loop(0, NPHASE)
        def _(ph):
            srow = (c * NSUB + s) * nchunks + ph * cpp
            drow = s * nchunks + ph * cpp
            pltpu.sync_copy(src_hbm.at[pl.ds(srow, cpp)], sidx)
            pltpu.sync_copy(dst_hbm.at[pl.ds(drow, cpp)], didx)

            pltpu.async_copy(x_hbm.at[sidx.at[0]], buf0, sem0)

            @pl.loop(0, npairs)
            def _(p):
                i0 = 2 * p
                pltpu.async_copy(x_hbm.at[sidx.at[i0 + 1]], buf1, sem1)
                pltpu.make_async_copy(x_hbm.at[sidx.at[i0]], buf0, sem0).wait()
                pltpu.sync_copy(buf0, acc.at[didx.at[i0]], add=True)

                @pl.when(i0 + 2 < cpp)
                def _():
                    pltpu.async_copy(x_hbm.at[sidx.at[i0 + 2]], buf0, sem0)

                pltpu.make_async_copy(x_hbm.at[sidx.at[i0 + 1]], buf1,
                                      sem1).wait()
                pltpu.sync_copy(buf1, acc.at[didx.at[i0 + 1]], add=True)

        plsc.subcore_barrier()
        out0 = c * ACC_ROWS + s * ZERO_ROWS
        pltpu.sync_copy(acc.at[pl.ds(s * ZERO_ROWS, ZERO_ROWS)],
                        out_hbm.at[pl.ds(out0, ZERO_ROWS)])

    return sc_kernel(x_flat, src2d, dst2d, zeros_blk)


def _mlp_block(h, a, W1, b1, W2, b2):
    u = h[...].reshape(-1, D) + jnp.concatenate([a[0], a[1]], axis=1)
    t = jnp.dot(u, W1[...], preferred_element_type=jnp.float32,
                precision=lax.Precision.DEFAULT) + b1[...]
    t = jnp.maximum(t, 0.0)
    v = jnp.dot(t, W2[...], preferred_element_type=jnp.float32,
                precision=lax.Precision.DEFAULT) + b2[...]
    return jnp.maximum(v, 0.0)


def _tc_layer_body(h, a, W1, b1, W2, b2, o, pool):
    v = _mlp_block(h, a, W1, b1, W2, b2)
    o[...] = v.reshape(-1, HALF)

    @pl.when(pl.program_id(0) == 0)
    def _():
        pool[...] = jnp.zeros_like(pool)

    pool[...] += jnp.sum(v, axis=0, keepdims=True)


_TC_ROWS = 1000


def _layer_in_specs():
    return [
        pl.BlockSpec((2 * _TC_ROWS, HALF), lambda i: (i, 0)),
        pl.BlockSpec((2, _TC_ROWS, HALF), lambda i: (0, i, 0)),
        pl.BlockSpec((D, D), lambda i: (0, 0)),
        pl.BlockSpec((1, D), lambda i: (0, 0)),
        pl.BlockSpec((D, D), lambda i: (0, 0)),
        pl.BlockSpec((1, D), lambda i: (0, 0)),
    ]


def _tc_layer(h_s, agg_s, W1, b1, W2, b2):
    return pl.pallas_call(
        _tc_layer_body,
        grid=(N // _TC_ROWS,),
        in_specs=_layer_in_specs(),
        out_specs=[
            pl.BlockSpec((2 * _TC_ROWS, HALF), lambda i: (i, 0)),
            pl.BlockSpec((1, D), lambda i: (0, 0)),
        ],
        out_shape=[
            jax.ShapeDtypeStruct((2 * N, HALF), jnp.float32),
            jax.ShapeDtypeStruct((1, D), jnp.float32),
        ],
    )(h_s, agg_s, W1, b1.reshape(1, D), W2, b2.reshape(1, D))


def _tc_layer3_body(h, a, W1, b1, W2, b2, p1, p2, lw1, lb1, lw2, lb2,
                    pool, o):
    v = _mlp_block(h, a, W1, b1, W2, b2)

    @pl.when(pl.program_id(0) == 0)
    def _():
        pool[...] = jnp.zeros_like(pool)

    pool[...] += jnp.sum(v, axis=0, keepdims=True)

    @pl.when(pl.program_id(0) == N // _TC_ROWS - 1)
    def _():
        p = jnp.concatenate([p1[...], p2[...], pool[...]], axis=1)
        hh = jnp.dot(p, lw1[...], preferred_element_type=jnp.float32,
                     precision=lax.Precision.DEFAULT) + lb1[...]
        hh = jnp.maximum(hh, 0.0)
        o[...] = jnp.dot(hh, lw2[...], preferred_element_type=jnp.float32,
                         precision=lax.Precision.DEFAULT) + lb2[...]


def _tc_layer3(h_s, agg_s, W1, b1, W2, b2, p1, p2,
               lin1_W, lin1_b, lin2_Wp, lin2_bp):
    return pl.pallas_call(
        _tc_layer3_body,
        grid=(N // _TC_ROWS,),
        in_specs=_layer_in_specs() + [
            pl.BlockSpec((1, D), lambda i: (0, 0)),
            pl.BlockSpec((1, D), lambda i: (0, 0)),
            pl.BlockSpec((3 * D, 3 * D), lambda i: (0, 0)),
            pl.BlockSpec((1, 3 * D), lambda i: (0, 0)),
            pl.BlockSpec((3 * D, HALF), lambda i: (0, 0)),
            pl.BlockSpec((1, HALF), lambda i: (0, 0)),
        ],
        out_specs=[
            pl.BlockSpec((1, D), lambda i: (0, 0)),
            pl.BlockSpec((1, HALF), lambda i: (0, 0)),
        ],
        out_shape=[
            jax.ShapeDtypeStruct((1, D), jnp.float32),
            jax.ShapeDtypeStruct((1, HALF), jnp.float32),
        ],
    )(h_s, agg_s, W1, b1.reshape(1, D), W2, b2.reshape(1, D), p1, p2,
      lin1_W, lin1_b.reshape(1, 3 * D), lin2_Wp, lin2_bp)


def kernel(x, edge_index, conv1_W1, conv1_b1, conv1_W2, conv1_b2,
           conv2_W1, conv2_b1, conv2_W2, conv2_b2,
           conv3_W1, conv3_b1, conv3_W2, conv3_b2,
           lin1_W, lin1_b, lin2_W, lin2_b):
    src = edge_index[0]
    dst = edge_index[1]
    E = src.shape[0]
    epc = 2 * NPHASE * NSUB * CHUNK
    e_pad = ((E + epc - 1) // epc) * epc
    pad = e_pad - E
    src_p = jnp.concatenate([src, jnp.zeros((pad,), jnp.int32)])
    dst_p = jnp.concatenate([dst, jnp.full((pad,), N, jnp.int32)])
    src_e = 2 * src_p
    src2d = jnp.concatenate([src_e, src_e + 1]).reshape(-1, CHUNK)
    dst2d = dst_p.reshape(-1, CHUNK)
    zeros_blk = jnp.zeros((ZERO_ROWS, HALF), jnp.float32)

    h = x.reshape(2 * N, HALF)

    pools = []
    for (W1, b1, W2, b2) in (
        (conv1_W1, conv1_b1, conv1_W2, conv1_b2),
        (conv2_W1, conv2_b1, conv2_W2, conv2_b2),
    ):
        agg = _sc_scatter_add(h, src2d, dst2d, zeros_blk, e_pad)
        h, p = _tc_layer(h, agg.reshape(2, ACC_ROWS, HALF), W1, b1, W2, b2)
        pools.append(p)

    C = lin2_W.shape[1]
    lin2_Wp = jnp.zeros((lin2_W.shape[0], HALF), jnp.float32).at[:, :C].set(lin2_W)
    lin2_bp = jnp.zeros((1, HALF), jnp.float32).at[:, :C].set(lin2_b)
    agg = _sc_scatter_add(h, src2d, dst2d, zeros_blk, e_pad)
    _, out = _tc_layer3(h, agg.reshape(2, ACC_ROWS, HALF),
                        conv3_W1, conv3_b1, conv3_W2, conv3_b2,
                        pools[0], pools[1], lin1_W, lin1_b, lin2_Wp, lin2_bp)
    return out[:, :C]


# speedup vs baseline: 3.2420x; 1.0038x over previous
import functools

import jax
import jax.numpy as jnp
from jax import lax
from jax.experimental import pallas as pl
from jax.experimental.pallas import tpu as pltpu
from jax.experimental.pallas import tpu_sc as plsc

N = 10000
D = 256
HALF = 128
NSUB = 16
CHUNK = 128
NPHASE = 2
ACC_ROWS = 10240
ZERO_ROWS = ACC_ROWS // NSUB


def _sc_scatter_add(x_flat, src2d, dst2d, zeros_blk, e_pad):
    nchunks = e_pad // (NSUB * CHUNK)
    cpp = nchunks // NPHASE
    npairs = cpp // 2

    mesh = plsc.VectorSubcoreMesh(core_axis_name="c", subcore_axis_name="s")

    @functools.partial(
        pl.kernel,
        out_type=jax.ShapeDtypeStruct((2 * ACC_ROWS, HALF), jnp.float32),
        mesh=mesh,
        scratch_types=[
            pltpu.VMEM_SHARED((ACC_ROWS, HALF), jnp.float32),
            pltpu.VMEM((cpp, CHUNK), jnp.int32),
            pltpu.VMEM((cpp, CHUNK), jnp.int32),
            pltpu.VMEM((CHUNK, HALF), jnp.float32),
            pltpu.VMEM((CHUNK, HALF), jnp.float32),
            pltpu.SemaphoreType.DMA,
            pltpu.SemaphoreType.DMA,
            pltpu.SemaphoreType.DMA,
        ],
    )
    def sc_kernel(x_hbm, src_hbm, dst_hbm, zero_hbm, out_hbm,
                  acc, sidx, didx, buf0, buf1, sem0, sem1, semz):
        c = lax.axis_index("c")
        s = lax.axis_index("s")
        pltpu.async_copy(zero_hbm, acc.at[pl.ds(s * ZERO_ROWS, ZERO_ROWS)],
                         semz)

        @pl.loop(0, NPHASE)
        def _(ph):
            srow = (c * NSUB + s) * nchunks + ph * cpp
            drow = s * nchunks + ph * cpp
            pltpu.sync_copy(src_hbm.at[pl.ds(srow, cpp)], sidx)
            pltpu.sync_copy(dst_hbm.at[pl.ds(drow, cpp)], didx)

            pltpu.async_copy(x_hbm.at[sidx.at[0]], buf0, sem0)

            @pl.when(ph == 0)
            def _():
                pltpu.make_async_copy(
                    zero_hbm, acc.at[pl.ds(s * ZERO_ROWS, ZERO_ROWS)],
                    semz).wait()
                plsc.subcore_barrier()

            @pl.loop(0, npairs)
            def _(p):
                i0 = 2 * p
                pltpu.async_copy(x_hbm.at[sidx.at[i0 + 1]], buf1, sem1)
                pltpu.make_async_copy(x_hbm.at[sidx.at[i0]], buf0, sem0).wait()
                pltpu.sync_copy(buf0, acc.at[didx.at[i0]], add=True)

                @pl.when(i0 + 2 < cpp)
                def _():
                    pltpu.async_copy(x_hbm.at[sidx.at[i0 + 2]], buf0, sem0)

                pltpu.make_async_copy(x_hbm.at[sidx.at[i0 + 1]], buf1,
                                      sem1).wait()
                pltpu.sync_copy(buf1, acc.at[didx.at[i0 + 1]], add=True)

        plsc.subcore_barrier()
        out0 = c * ACC_ROWS + s * ZERO_ROWS
        pltpu.sync_copy(acc.at[pl.ds(s * ZERO_ROWS, ZERO_ROWS)],
                        out_hbm.at[pl.ds(out0, ZERO_ROWS)])

    return sc_kernel(x_flat, src2d, dst2d, zeros_blk)


def _mlp_block(h, a, W1, b1, W2, b2):
    u = h[...].reshape(-1, D) + jnp.concatenate([a[0], a[1]], axis=1)
    t = jnp.dot(u, W1[...], preferred_element_type=jnp.float32,
                precision=lax.Precision.DEFAULT) + b1[...]
    t = jnp.maximum(t, 0.0)
    v = jnp.dot(t, W2[...], preferred_element_type=jnp.float32,
                precision=lax.Precision.DEFAULT) + b2[...]
    return jnp.maximum(v, 0.0)


def _tc_layer_body(h, a, W1, b1, W2, b2, o, pool):
    v = _mlp_block(h, a, W1, b1, W2, b2)
    o[...] = v.reshape(-1, HALF)

    @pl.when(pl.program_id(0) == 0)
    def _():
        pool[...] = jnp.zeros_like(pool)

    pool[...] += jnp.sum(v, axis=0, keepdims=True)


_TC_ROWS = 1000


def _layer_in_specs():
    return [
        pl.BlockSpec((2 * _TC_ROWS, HALF), lambda i: (i, 0)),
        pl.BlockSpec((2, _TC_ROWS, HALF), lambda i: (0, i, 0)),
        pl.BlockSpec((D, D), lambda i: (0, 0)),
        pl.BlockSpec((1, D), lambda i: (0, 0)),
        pl.BlockSpec((D, D), lambda i: (0, 0)),
        pl.BlockSpec((1, D), lambda i: (0, 0)),
    ]


def _tc_layer(h_s, agg_s, W1, b1, W2, b2):
    return pl.pallas_call(
        _tc_layer_body,
        grid=(N // _TC_ROWS,),
        in_specs=_layer_in_specs(),
        out_specs=[
            pl.BlockSpec((2 * _TC_ROWS, HALF), lambda i: (i, 0)),
            pl.BlockSpec((1, D), lambda i: (0, 0)),
        ],
        out_shape=[
            jax.ShapeDtypeStruct((2 * N, HALF), jnp.float32),
            jax.ShapeDtypeStruct((1, D), jnp.float32),
        ],
    )(h_s, agg_s, W1, b1.reshape(1, D), W2, b2.reshape(1, D))


def _tc_layer3_body(h, a, W1, b1, W2, b2, p1, p2, lw1, lb1, lw2, lb2,
                    pool, o):
    v = _mlp_block(h, a, W1, b1, W2, b2)

    @pl.when(pl.program_id(0) == 0)
    def _():
        pool[...] = jnp.zeros_like(pool)

    pool[...] += jnp.sum(v, axis=0, keepdims=True)

    @pl.when(pl.program_id(0) == N // _TC_ROWS - 1)
    def _():
        p = jnp.concatenate([p1[...], p2[...], pool[...]], axis=1)
        hh = jnp.dot(p, lw1[...], preferred_element_type=jnp.float32,
                     precision=lax.Precision.DEFAULT) + lb1[...]
        hh = jnp.maximum(hh, 0.0)
        o[...] = jnp.dot(hh, lw2[...], preferred_element_type=jnp.float32,
                         precision=lax.Precision.DEFAULT) + lb2[...]


def _tc_layer3(h_s, agg_s, W1, b1, W2, b2, p1, p2,
               lin1_W, lin1_b, lin2_Wp, lin2_bp):
    return pl.pallas_call(
        _tc_layer3_body,
        grid=(N // _TC_ROWS,),
        in_specs=_layer_in_specs() + [
            pl.BlockSpec((1, D), lambda i: (0, 0)),
            pl.BlockSpec((1, D), lambda i: (0, 0)),
            pl.BlockSpec((3 * D, 3 * D), lambda i: (0, 0)),
            pl.BlockSpec((1, 3 * D), lambda i: (0, 0)),
            pl.BlockSpec((3 * D, HALF), lambda i: (0, 0)),
            pl.BlockSpec((1, HALF), lambda i: (0, 0)),
        ],
        out_specs=[
            pl.BlockSpec((1, D), lambda i: (0, 0)),
            pl.BlockSpec((1, HALF), lambda i: (0, 0)),
        ],
        out_shape=[
            jax.ShapeDtypeStruct((1, D), jnp.float32),
            jax.ShapeDtypeStruct((1, HALF), jnp.float32),
        ],
    )(h_s, agg_s, W1, b1.reshape(1, D), W2, b2.reshape(1, D), p1, p2,
      lin1_W, lin1_b.reshape(1, 3 * D), lin2_Wp, lin2_bp)


def kernel(x, edge_index, conv1_W1, conv1_b1, conv1_W2, conv1_b2,
           conv2_W1, conv2_b1, conv2_W2, conv2_b2,
           conv3_W1, conv3_b1, conv3_W2, conv3_b2,
           lin1_W, lin1_b, lin2_W, lin2_b):
    src = edge_index[0]
    dst = edge_index[1]
    E = src.shape[0]
    epc = 2 * NPHASE * NSUB * CHUNK
    e_pad = ((E + epc - 1) // epc) * epc
    pad = e_pad - E
    src_p = jnp.concatenate([src, jnp.zeros((pad,), jnp.int32)])
    dst_p = jnp.concatenate([dst, jnp.full((pad,), N, jnp.int32)])
    src_e = 2 * src_p
    src2d = jnp.concatenate([src_e, src_e + 1]).reshape(-1, CHUNK)
    dst2d = dst_p.reshape(-1, CHUNK)
    zeros_blk = jnp.zeros((ZERO_ROWS, HALF), jnp.float32)

    h = x.reshape(2 * N, HALF)

    pools = []
    for (W1, b1, W2, b2) in (
        (conv1_W1, conv1_b1, conv1_W2, conv1_b2),
        (conv2_W1, conv2_b1, conv2_W2, conv2_b2),
    ):
        agg = _sc_scatter_add(h, src2d, dst2d, zeros_blk, e_pad)
        h, p = _tc_layer(h, agg.reshape(2, ACC_ROWS, HALF), W1, b1, W2, b2)
        pools.append(p)

    C = lin2_W.shape[1]
    lin2_Wp = jnp.zeros((lin2_W.shape[0], HALF), jnp.float32).at[:, :C].set(lin2_W)
    lin2_bp = jnp.zeros((1, HALF), jnp.float32).at[:, :C].set(lin2_b)
    agg = _sc_scatter_add(h, src2d, dst2d, zeros_blk, e_pad)
    _, out = _tc_layer3(h, agg.reshape(2, ACC_ROWS, HALF),
                        conv3_W1, conv3_b1, conv3_W2, conv3_b2,
                        pools[0], pools[1], lin1_W, lin1_b, lin2_Wp, lin2_bp)
    return out[:, :C]
